# baseline (speedup 1.0000x reference)
import functools

import jax
import jax.numpy as jnp
from jax import lax
from jax.experimental import pallas as pl
from jax.experimental.pallas import tpu as pltpu
from jax.experimental.pallas import tpu_sc as plsc

NC = 2
NS = 16
NW = NC * NS
LANES = 16

VOCAB = 100000
MAXLEN = 200
EMBED = 64
EPAD = 128
BATCH = 4096

BPW = BATCH // NW
TB = BPW // LANES
assert BATCH % NW == 0 and BPW == 128 and MAXLEN % 2 == 0


def _emb_body(xt_hbm, tok_hbm, pos_hbm, out_hbm,
              idx_v, pos_v, rows0, rows1, ob0, ob1,
              gsem0, gsem1, osem0, osem1):
    rows = (rows0, rows1)
    ob = (ob0, ob1)
    gsem = (gsem0, gsem1)
    osem = (osem0, osem1)
    wid = lax.axis_index("s") * NC + lax.axis_index("c")
    b0 = wid * BPW
    pltpu.sync_copy(pos_hbm, pos_v)
    pltpu.sync_copy(xt_hbm.at[:, pl.ds(b0, BPW)], idx_v)
    iota = lax.iota(jnp.int32, LANES)
    zero = iota * 0
    trow = [t * LANES + iota for t in range(TB)]

    def start_gather(l, b):
        pltpu.async_copy(tok_hbm.at[idx_v.at[l]], rows[b], gsem[b])

    def wait_gather(b):
        pltpu.make_async_copy(tok_hbm.at[idx_v.at[0]], rows[b], gsem[b]).wait()

    def transpose_add(l, b):
        rv, ov = rows[b], ob[b]
        sl = zero + l

        def _half(h, carry):
            e0 = h * (EMBED // 2)
            for e in range(EMBED // 2):
                se = zero + (e0 + e)
                pe = plsc.load_gather(pos_v, [sl, se])
                for t in range(TB):
                    v = plsc.load_gather(rv, [trow[t], se])
                    ov[e0 + e, pl.ds(t * LANES, LANES)] = v + pe
            return carry

        lax.fori_loop(0, 2, _half, None)

    def start_write(l, b):
        pltpu.async_copy(ob[b], out_hbm.at[l, :, pl.ds(b0, BPW)], osem[b])

    def wait_write(b):
        pltpu.make_async_copy(ob[b], out_hbm.at[0, :, pl.ds(b0, BPW)], osem[b]).wait()

    start_gather(0, 0)
    wait_gather(0)
    start_gather(1, 1)
    transpose_add(0, 0)
    start_write(0, 0)

    @pl.loop(0, (MAXLEN - 2) // 2)
    def _steady(t):
        for b in (1, 0):
            l = 1 + 2 * t + (0 if b == 1 else 1)
            wait_gather(b)
            obuf = 1 - b
            wait_write(obuf)
            start_gather(l + 1, obuf)
            transpose_add(l, b)
            start_write(l, b)

    wait_gather(1)
    transpose_add(MAXLEN - 1, 1)
    start_write(MAXLEN - 1, 1)
    wait_write(0)
    wait_write(1)


_emb = functools.partial(
    pl.kernel,
    out_type=jax.ShapeDtypeStruct((MAXLEN, EMBED, BATCH), jnp.float32),
    mesh=plsc.VectorSubcoreMesh(core_axis_name="c", subcore_axis_name="s"),
    scratch_types=[
        pltpu.VMEM((MAXLEN, BPW), jnp.int32),
        pltpu.VMEM((MAXLEN, EPAD), jnp.float32),
        pltpu.VMEM((BPW, EPAD), jnp.float32),
        pltpu.VMEM((BPW, EPAD), jnp.float32),
        pltpu.VMEM((EMBED, BPW), jnp.float32),
        pltpu.VMEM((EMBED, BPW), jnp.float32),
        pltpu.SemaphoreType.DMA,
        pltpu.SemaphoreType.DMA,
        pltpu.SemaphoreType.DMA,
        pltpu.SemaphoreType.DMA,
    ],
    compiler_params=pltpu.CompilerParams(
        use_tc_tiling_on_sc=True, needs_layout_passes=False),
)(_emb_body)


def kernel(x, token_table, pos_table):
    xt = x.astype(jnp.int32).T
    tok_pad = jnp.pad(token_table, ((0, 0), (0, EPAD - EMBED)))
    pos_pad = jnp.pad(pos_table, ((0, 0), (0, EPAD - EMBED)))
    out_t = _emb(xt, tok_pad, pos_pad)
    return out_t.transpose(2, 0, 1)


# speedup vs baseline: 1.8149x; 1.8149x over previous
import functools

import jax
import jax.numpy as jnp
from jax import lax
from jax.experimental import pallas as pl
from jax.experimental.pallas import tpu as pltpu
from jax.experimental.pallas import tpu_sc as plsc

NC = 2
NS = 16
NW = NC * NS
LANES = 16

VOCAB = 100000
MAXLEN = 200
EMBED = 64
EPAD = 128
BATCH = 4096

BPW = BATCH // NW
TB = BPW // LANES
assert BATCH % NW == 0 and BPW == 128 and MAXLEN % 2 == 0


def _emb_body(xt_hbm, tok_hbm, pos_hbm, out_hbm,
              idx_v, pos_v, rows0, rows1, ob0, ob1,
              gsem0, gsem1, osem0, osem1):
    rows = (rows0, rows1)
    ob = (ob0, ob1)
    gsem = (gsem0, gsem1)
    osem = (osem0, osem1)
    wid = lax.axis_index("s") * NC + lax.axis_index("c")
    b0 = wid * BPW
    pltpu.sync_copy(pos_hbm, pos_v)
    pltpu.sync_copy(xt_hbm.at[:, pl.ds(b0, BPW)], idx_v)
    iota = lax.iota(jnp.int32, LANES)
    zero = iota * 0
    trow = [t * LANES + iota for t in range(TB)]

    def start_gather(l, b):
        pltpu.async_copy(tok_hbm.at[idx_v.at[l]], rows[b], gsem[b])

    def wait_gather(b):
        pltpu.make_async_copy(tok_hbm.at[idx_v.at[0]], rows[b], gsem[b]).wait()

    def transpose_add(l, b):
        rv, ov = rows[b], ob[b]
        sl = zero + l

        @plsc.parallel_loop(0, EMBED, unroll=8)
        def _col(e):
            se = zero + e
            pe = plsc.load_gather(pos_v, [sl, se])
            for t in range(TB):
                v = plsc.load_gather(rv, [trow[t], se])
                ov[e, pl.ds(t * LANES, LANES)] = v + pe

    def start_write(l, b):
        pltpu.async_copy(ob[b], out_hbm.at[l, :, pl.ds(b0, BPW)], osem[b])

    def wait_write(b):
        pltpu.make_async_copy(ob[b], out_hbm.at[0, :, pl.ds(b0, BPW)], osem[b]).wait()

    start_gather(0, 0)
    wait_gather(0)
    start_gather(1, 1)
    transpose_add(0, 0)
    start_write(0, 0)

    @pl.loop(0, (MAXLEN - 2) // 2)
    def _steady(t):
        for b in (1, 0):
            l = 1 + 2 * t + (0 if b == 1 else 1)
            wait_gather(b)
            obuf = 1 - b
            wait_write(obuf)
            start_gather(l + 1, obuf)
            transpose_add(l, b)
            start_write(l, b)

    wait_gather(1)
    transpose_add(MAXLEN - 1, 1)
    start_write(MAXLEN - 1, 1)
    wait_write(0)
    wait_write(1)


_emb = functools.partial(
    pl.kernel,
    out_type=jax.ShapeDtypeStruct((MAXLEN, EMBED, BATCH), jnp.float32),
    mesh=plsc.VectorSubcoreMesh(core_axis_name="c", subcore_axis_name="s"),
    scratch_types=[
        pltpu.VMEM((MAXLEN, BPW), jnp.int32),
        pltpu.VMEM((MAXLEN, EPAD), jnp.float32),
        pltpu.VMEM((BPW, EPAD), jnp.float32),
        pltpu.VMEM((BPW, EPAD), jnp.float32),
        pltpu.VMEM((EMBED, BPW), jnp.float32),
        pltpu.VMEM((EMBED, BPW), jnp.float32),
        pltpu.SemaphoreType.DMA,
        pltpu.SemaphoreType.DMA,
        pltpu.SemaphoreType.DMA,
        pltpu.SemaphoreType.DMA,
    ],
    compiler_params=pltpu.CompilerParams(
        use_tc_tiling_on_sc=True, needs_layout_passes=False),
)(_emb_body)


def kernel(x, token_table, pos_table):
    xt = x.astype(jnp.int32).T
    tok_pad = jnp.pad(token_table, ((0, 0), (0, EPAD - EMBED)))
    pos_pad = jnp.pad(pos_table, ((0, 0), (0, EPAD - EMBED)))
    out_t = _emb(xt, tok_pad, pos_pad)
    return out_t.transpose(2, 0, 1)


# speedup vs baseline: 3.8590x; 2.1263x over previous
import functools

import jax
import jax.numpy as jnp
from jax import lax
from jax.experimental import pallas as pl
from jax.experimental.pallas import tpu as pltpu
from jax.experimental.pallas import tpu_sc as plsc

NC = 2
NS = 16
NW = NC * NS
LANES = 16

VOCAB = 100000
MAXLEN = 200
EMBED = 64
EPAD = 128
BATCH = 4096

BPW = BATCH // NW
TB = BPW // LANES
assert BATCH % NW == 0 and BPW == 128 and MAXLEN % 2 == 0


SSTR = 73


def _emb_body(xt_hbm, tok_hbm, pos_hbm, out_hbm,
              idx_v, pos_v, rows0, rows1, stg, ob0, ob1,
              gsem0, gsem1, osem0, osem1):
    rows = (rows0, rows1)
    ob = (ob0, ob1)
    gsem = (gsem0, gsem1)
    osem = (osem0, osem1)
    wid = lax.axis_index("s") * NC + lax.axis_index("c")
    b0 = wid * BPW
    pltpu.sync_copy(pos_hbm, pos_v)
    pltpu.sync_copy(xt_hbm.at[:, pl.ds(b0, BPW)], idx_v)
    iota = lax.iota(jnp.int32, LANES)
    zero = iota * 0
    tcol = [iota * SSTR + t * LANES * SSTR for t in range(TB)]

    def start_gather(l, b):
        pltpu.async_copy(tok_hbm.at[idx_v.at[l]], rows[b], gsem[b])

    def wait_gather(b):
        pltpu.make_async_copy(tok_hbm.at[idx_v.at[0]], rows[b], gsem[b]).wait()

    def transpose_add(l, b):
        rv, ov = rows[b], ob[b]
        sl = zero + l

        @plsc.parallel_loop(0, BPW, unroll=8)
        def _stage(t):
            base = t * SSTR
            for k in range(EMBED // LANES):
                stg[pl.ds(base + k * LANES, LANES)] = rv[t, pl.ds(k * LANES, LANES)]

        @plsc.parallel_loop(0, EMBED, unroll=8)
        def _col(e):
            se = zero + e
            pe = plsc.load_gather(pos_v, [sl, se])
            for t in range(TB):
                v = plsc.load_gather(stg, [tcol[t] + e])
                ov[e, pl.ds(t * LANES, LANES)] = v + pe

    def start_write(l, b):
        pltpu.async_copy(ob[b], out_hbm.at[l, :, pl.ds(b0, BPW)], osem[b])

    def wait_write(b):
        pltpu.make_async_copy(ob[b], out_hbm.at[0, :, pl.ds(b0, BPW)], osem[b]).wait()

    start_gather(0, 0)
    wait_gather(0)
    start_gather(1, 1)
    transpose_add(0, 0)
    start_write(0, 0)

    @pl.loop(0, (MAXLEN - 2) // 2)
    def _steady(t):
        for b in (1, 0):
            l = 1 + 2 * t + (0 if b == 1 else 1)
            wait_gather(b)
            obuf = 1 - b
            wait_write(obuf)
            start_gather(l + 1, obuf)
            transpose_add(l, b)
            start_write(l, b)

    wait_gather(1)
    transpose_add(MAXLEN - 1, 1)
    start_write(MAXLEN - 1, 1)
    wait_write(0)
    wait_write(1)


_emb = functools.partial(
    pl.kernel,
    out_type=jax.ShapeDtypeStruct((MAXLEN, EMBED, BATCH), jnp.float32),
    mesh=plsc.VectorSubcoreMesh(core_axis_name="c", subcore_axis_name="s"),
    scratch_types=[
        pltpu.VMEM((MAXLEN, BPW), jnp.int32),
        pltpu.VMEM((MAXLEN, EPAD), jnp.float32),
        pltpu.VMEM((BPW, EPAD), jnp.float32),
        pltpu.VMEM((BPW, EPAD), jnp.float32),
        pltpu.VMEM((BPW * SSTR,), jnp.float32),
        pltpu.VMEM((EMBED, BPW), jnp.float32),
        pltpu.VMEM((EMBED, BPW), jnp.float32),
        pltpu.SemaphoreType.DMA,
        pltpu.SemaphoreType.DMA,
        pltpu.SemaphoreType.DMA,
        pltpu.SemaphoreType.DMA,
    ],
    compiler_params=pltpu.CompilerParams(
        use_tc_tiling_on_sc=True, needs_layout_passes=False),
)(_emb_body)


def kernel(x, token_table, pos_table):
    xt = x.astype(jnp.int32).T
    tok_pad = jnp.pad(token_table, ((0, 0), (0, EPAD - EMBED)))
    pos_pad = jnp.pad(pos_table, ((0, 0), (0, EPAD - EMBED)))
    out_t = _emb(xt, tok_pad, pos_pad)
    return out_t.transpose(2, 0, 1)


# speedup vs baseline: 5.0182x; 1.3004x over previous
import functools

import jax
import jax.numpy as jnp
from jax import lax
from jax.experimental import pallas as pl
from jax.experimental.pallas import tpu as pltpu
from jax.experimental.pallas import tpu_sc as plsc

NC = 2
NS = 16
NW = NC * NS
LANES = 16

VOCAB = 100000
MAXLEN = 200
EMBED = 64
EPAD = 128
BATCH = 4096

BPW = BATCH // NW
TB = BPW // LANES
KCH = EMBED // LANES
assert BATCH % NW == 0 and BPW == 128 and MAXLEN == 200

SSTR = 73


def _emb_body(xt_hbm, tok_hbm, pos_hbm, out_hbm,
              idx_v, pos_v, rows0, rows1, rows2, stg, ob0, ob1,
              gsem0, gsem1, gsem2, osem0, osem1):
    rows = (rows0, rows1, rows2)
    ob = (ob0, ob1)
    gsem = (gsem0, gsem1, gsem2)
    osem = (osem0, osem1)
    wid = lax.axis_index("s") * NC + lax.axis_index("c")
    b0 = wid * BPW
    pltpu.sync_copy(pos_hbm, pos_v)
    pltpu.sync_copy(xt_hbm.at[:, pl.ds(b0, BPW)], idx_v)
    iota = lax.iota(jnp.int32, LANES)
    zero = iota * 0
    tcol = [iota * SSTR + t * LANES * SSTR for t in range(TB)]

    def sg(l, b):
        pltpu.async_copy(tok_hbm.at[idx_v.at[l]], rows[b], gsem[b])

    def wg(b):
        pltpu.make_async_copy(tok_hbm.at[idx_v.at[0]], rows[b], gsem[b]).wait()

    def transpose_add(l, b, w):
        rv, ov = rows[b], ob[w]
        sl64 = l * EMBED

        @plsc.parallel_loop(0, BPW, unroll=8)
        def _stage(t):
            base = t * SSTR
            for k in range(KCH):
                stg[pl.ds(base + k * LANES, LANES)] = rv[t, pl.ds(k * LANES, LANES)]

        @plsc.parallel_loop(0, EMBED, unroll=8)
        def _col(e):
            pe = plsc.load_gather(pos_v, [zero + (sl64 + e)])
            for t in range(TB):
                v = plsc.load_gather(stg, [tcol[t] + e])
                ov[e, pl.ds(t * LANES, LANES)] = v + pe

    def sw(l, w):
        pltpu.async_copy(ob[w], out_hbm.at[l, :, pl.ds(b0, BPW)], osem[w])

    def ww(w):
        pltpu.make_async_copy(ob[w], out_hbm.at[0, :, pl.ds(b0, BPW)], osem[w]).wait()

    sg(0, 0)
    sg(1, 1)
    wg(0)
    sg(2, 2)
    transpose_add(0, 0, 0)
    sw(0, 0)
    wg(1)
    sg(3, 0)
    transpose_add(1, 1, 1)
    sw(1, 1)
    wg(2)
    sg(4, 1)
    ww(0)
    transpose_add(2, 2, 0)
    sw(2, 0)

    @pl.loop(0, (MAXLEN - 8) // 6)
    def _steady(t):
        for j in range(6):
            l = 3 + 6 * t + j
            b = j % 3
            w = (j + 1) % 2
            wg(b)
            ww(w)
            sg(l + 2, (j + 2) % 3)
            transpose_add(l, b, w)
            sw(l, w)

    for j, (b, w) in enumerate([(0, 1), (1, 0), (2, 1), (0, 0), (1, 1)]):
        l = 195 + j
        wg(b)
        ww(w)
        if l + 2 < MAXLEN:
            sg(l + 2, (b + 2) % 3)
        transpose_add(l, b, w)
        sw(l, w)
    ww(0)
    ww(1)


_emb = functools.partial(
    pl.kernel,
    out_type=jax.ShapeDtypeStruct((MAXLEN, EMBED, BATCH), jnp.float32),
    mesh=plsc.VectorSubcoreMesh(core_axis_name="c", subcore_axis_name="s"),
    scratch_types=[
        pltpu.VMEM((MAXLEN, BPW), jnp.int32),
        pltpu.VMEM((MAXLEN * EMBED,), jnp.float32),
        pltpu.VMEM((BPW, EPAD), jnp.float32),
        pltpu.VMEM((BPW, EPAD), jnp.float32),
        pltpu.VMEM((BPW, EPAD), jnp.float32),
        pltpu.VMEM((BPW * SSTR,), jnp.float32),
        pltpu.VMEM((EMBED, BPW), jnp.float32),
        pltpu.VMEM((EMBED, BPW), jnp.float32),
        pltpu.SemaphoreType.DMA,
        pltpu.SemaphoreType.DMA,
        pltpu.SemaphoreType.DMA,
        pltpu.SemaphoreType.DMA,
        pltpu.SemaphoreType.DMA,
    ],
    compiler_params=pltpu.CompilerParams(
        use_tc_tiling_on_sc=True, needs_layout_passes=False),
)(_emb_body)


def kernel(x, token_table, pos_table):
    xt = x.astype(jnp.int32).T
    tok_pad = jnp.pad(token_table, ((0, 0), (0, EPAD - EMBED)))
    pos_flat = pos_table.reshape(-1)
    out_t = _emb(xt, tok_pad, pos_flat)
    return out_t.transpose(2, 0, 1)
